# baseline (speedup 1.0000x reference)
import functools
import numpy as np
import jax
import jax.numpy as jnp
from jax import lax
from jax.experimental import pallas as pl
from jax.experimental.pallas import tpu as pltpu
from jax.experimental.pallas import tpu_sc as plsc

N, E, F, T, HID, G, HLS, LAYERS = 10000, 320000, 128, 2, 128, 64, 32, 2
FT = HID // T
_DEG_HIST = np.array([0, 100, 500, 1500, 2500, 2400, 1500, 800, 400, 200, 100],
                     dtype=np.float64)
_LOG_AVG = float((np.log(np.arange(len(_DEG_HIST)) + 1.0) * _DEG_HIST).sum()
                 / _DEG_HIST.sum())

_NC, _NS = 2, 16
_NW = _NC * _NS
_NV = 64
_ROWS = 157
_RL = _ROWS * 128
_DPAD = 160
_LCAP = 8192
_FLUSH = 2048
_SCCH = 4000
_BCH = 128
_OBUF = _FLUSH + 512 + 16
_BIG = 3.0e38

_mesh = plsc.VectorSubcoreMesh(core_axis_name="c", subcore_axis_name="s",
                               num_cores=_NC, num_subcores=_NS)


def _wid():
    return lax.axis_index("s") * _NC + lax.axis_index("c")


@functools.partial(
    pl.kernel,
    out_type=(
        jax.ShapeDtypeStruct((_NV * _LCAP,), jnp.int32),
        jax.ShapeDtypeStruct((_NV * _LCAP,), jnp.float32),
        jax.ShapeDtypeStruct((_NV * _LCAP,), jnp.float32),
        jax.ShapeDtypeStruct((_NV * 128,), jnp.int32),
    ),
    mesh=_mesh,
    scratch_types=[
        pltpu.VMEM((_SCCH,), jnp.int32),
        pltpu.VMEM((_SCCH,), jnp.int32),
        pltpu.VMEM((_SCCH,), jnp.float32),
        pltpu.VMEM((_SCCH,), jnp.float32),
        pltpu.VMEM((_OBUF,), jnp.int32),
        pltpu.VMEM((_OBUF,), jnp.float32),
        pltpu.VMEM((_OBUF,), jnp.float32),
        pltpu.VMEM((_OBUF,), jnp.int32),
        pltpu.VMEM((_OBUF,), jnp.float32),
        pltpu.VMEM((_OBUF,), jnp.float32),
        pltpu.VMEM((128,), jnp.int32),
    ],
    compiler_params=pltpu.CompilerParams(needs_layout_passes=False),
)
def _scan_kernel(dst_h, src_h, ea0_h, ea1_h,
                 opk_h, oea0_h, oea1_h, cnt_h,
                 dbuf, sbuf, e0buf, e1buf,
                 pk0, ea0b0, ea1b0, pk1, ea0b1, ea1b1, cstage):
    k = _wid()

    bufs = ((pk0, ea0b0, ea1b0), (pk1, ea0b1, ea1b1))

    def chunk(ci, carry):
        p0, nf0, p1, nf1 = carry
        base = ci * _SCCH
        pltpu.sync_copy(dst_h.at[pl.ds(base, _SCCH)], dbuf)
        pltpu.sync_copy(src_h.at[pl.ds(base, _SCCH)], sbuf)
        pltpu.sync_copy(ea0_h.at[pl.ds(base, _SCCH)], e0buf)
        pltpu.sync_copy(ea1_h.at[pl.ds(base, _SCCH)], e1buf)

        def vbody(i, ps):
            q0, q1 = ps
            off = i * 16
            d = dbuf[pl.ds(off, 16)]
            sub = lax.shift_right_logical(d, 5) & 1
            r = lax.shift_right_logical(d, 6)
            pk = sbuf[pl.ds(off, 16)] | (r << 14)
            e0 = e0buf[pl.ds(off, 16)]
            e1 = e1buf[pl.ds(off, 16)]
            manyk = (d & 31) == k
            m0 = manyk & (sub == 0)
            m1 = manyk & (sub == 1)
            c0 = m0.astype(jnp.int32)
            c1 = m1.astype(jnp.int32)
            i0 = plsc.cumsum(c0)
            i1 = plsc.cumsum(c1)
            x0 = q0 + i0 - c0
            x1 = q1 + i1 - c1
            plsc.store_scatter(pk0, [x0], pk, mask=m0)
            plsc.store_scatter(ea0b0, [x0], e0, mask=m0)
            plsc.store_scatter(ea1b0, [x0], e1, mask=m0)
            plsc.store_scatter(pk1, [x1], pk, mask=m1)
            plsc.store_scatter(ea0b1, [x1], e0, mask=m1)
            plsc.store_scatter(ea1b1, [x1], e1, mask=m1)
            return (q0 + i0[15], q1 + i1[15])

        p0, p1 = lax.fori_loop(0, _SCCH // 16, vbody, (p0, p1))

        def mkflush(sl):
            pkb, e0b, e1b = bufs[sl]

            def do_flush(c):
                p, nf = c
                fb = (2 * k + sl) * _LCAP + nf * _FLUSH
                pltpu.sync_copy(pkb.at[pl.ds(0, _FLUSH)],
                                opk_h.at[pl.ds(fb, _FLUSH)])
                pltpu.sync_copy(e0b.at[pl.ds(0, _FLUSH)],
                                oea0_h.at[pl.ds(fb, _FLUSH)])
                pltpu.sync_copy(e1b.at[pl.ds(0, _FLUSH)],
                                oea1_h.at[pl.ds(fb, _FLUSH)])
                rem = p - _FLUSH

                def mv(j, c2):
                    o = j * 16
                    pkb[pl.ds(o, 16)] = pkb[pl.ds(_FLUSH + o, 16)]
                    e0b[pl.ds(o, 16)] = e0b[pl.ds(_FLUSH + o, 16)]
                    e1b[pl.ds(o, 16)] = e1b[pl.ds(_FLUSH + o, 16)]
                    return c2
                lax.fori_loop(0, (rem + 15) // 16, mv, 0)
                return rem, nf + 1
            return do_flush

        p0, nf0 = lax.cond(p0 >= _FLUSH, mkflush(0), lambda c: c, (p0, nf0))
        p1, nf1 = lax.cond(p1 >= _FLUSH, mkflush(1), lambda c: c, (p1, nf1))
        return (p0, nf0, p1, nf1)

    p0, nf0, p1, nf1 = lax.fori_loop(0, E // _SCCH, chunk, (0, 0, 0, 0))

    for sl, (p, nf) in enumerate(((p0, nf0), (p1, nf1))):
        pkb, e0b, e1b = bufs[sl]
        fb = (2 * k + sl) * _LCAP + nf * _FLUSH
        pltpu.sync_copy(pkb.at[pl.ds(0, _FLUSH)], opk_h.at[pl.ds(fb, _FLUSH)])
        pltpu.sync_copy(e0b.at[pl.ds(0, _FLUSH)], oea0_h.at[pl.ds(fb, _FLUSH)])
        pltpu.sync_copy(e1b.at[pl.ds(0, _FLUSH)], oea1_h.at[pl.ds(fb, _FLUSH)])
        total = nf * _FLUSH + p
        for j in range(8):
            cstage[pl.ds(j * 16, 16)] = jnp.full((16,), total, jnp.int32)
        pltpu.sync_copy(cstage, cnt_h.at[pl.ds((2 * k + sl) * 128, 128)])


@functools.partial(
    pl.kernel,
    out_type=(jax.ShapeDtypeStruct((2 * 4 * _NV * _RL,), jnp.float32),
              jax.ShapeDtypeStruct((_NV * _ROWS * 16,), jnp.float32)),
    mesh=_mesh,
    scratch_types=[
        pltpu.VMEM((_RL,), jnp.float32),
        pltpu.VMEM((_RL,), jnp.float32),
        pltpu.VMEM((_RL,), jnp.float32),
        pltpu.VMEM((_RL,), jnp.float32),
        pltpu.VMEM((_BCH,), jnp.int32),
        pltpu.VMEM((_BCH,), jnp.int32),
        pltpu.VMEM((_BCH + 16,), jnp.int32),
        pltpu.VMEM((_BCH + 16,), jnp.float32),
        pltpu.VMEM((_BCH + 16,), jnp.float32),
        pltpu.VMEM((_BCH,), jnp.int32),
        pltpu.VMEM((_BCH,), jnp.float32),
        pltpu.VMEM((_BCH,), jnp.float32),
        pltpu.VMEM((_BCH, 128), jnp.float32),
        pltpu.VMEM((_BCH, 128), jnp.float32),
        pltpu.VMEM((256,), jnp.float32),
        pltpu.VMEM((128,), jnp.int32),
        pltpu.VMEM((_ROWS * 16,), jnp.float32),
        pltpu.SemaphoreType.DMA,
        pltpu.SemaphoreType.DMA,
        pltpu.SemaphoreType.DMA,
    ],
    compiler_params=pltpu.CompilerParams(needs_layout_passes=False),
)
def _accum_kernel(b0_h, b1_h, opk_h, oea0_h, oea1_h, cnt_h, wv_h, out_h,
                  deg_h, sum_r, sq_r, mn_r, mx_r, sbuf0, sbuf1, pkc, e0c, e1c,
                  pkn, e0n, e1n, rows0, rows1, wbuf, cstage, degl,
                  sem_l, sem_g0, sem_g1, *_unused):
    k = _wid()
    zf = jnp.zeros((16,), jnp.float32)
    bigf = jnp.full((16,), _BIG, jnp.float32)
    sbufs = (sbuf0, sbuf1)
    rowss = (rows0, rows1)
    sem_gs = (sem_g0, sem_g1)
    maxlb = _LCAP - _BCH

    for t, b_h in enumerate((b0_h, b1_h)):
        pltpu.sync_copy(wv_h.at[pl.ds(t * 256, 256)], wbuf)
        for sl in range(2):
            v = 2 * k + sl
            vbase = v * _LCAP
            pltpu.sync_copy(cnt_h.at[pl.ds(v * 128, 128)], cstage)
            cnt = cstage[pl.ds(0, 16)][0]
            nch = (cnt + _BCH - 1) // _BCH
            ngr = (nch + 1) // 2

            def initb(j, c):
                o = j * 16
                sum_r[pl.ds(o, 16)] = zf
                sq_r[pl.ds(o, 16)] = zf
                mn_r[pl.ds(o, 16)] = bigf
                mx_r[pl.ds(o, 16)] = -bigf
                return c
            lax.fori_loop(0, _RL // 16, initb, 0)
            if t == 0:
                def initd(j, c):
                    degl[pl.ds(j * 16, 16)] = zf
                    return c
                lax.fori_loop(0, _ROWS, initd, 0)

            def issue_lists(cb):
                lb = vbase + jnp.minimum(cb, maxlb)
                pltpu.async_copy(opk_h.at[pl.ds(lb, _BCH)], pkn, sem_l)
                pltpu.async_copy(oea0_h.at[pl.ds(lb, _BCH)], e0n, sem_l)
                pltpu.async_copy(oea1_h.at[pl.ds(lb, _BCH)], e1n, sem_l)

            def wait_lists():
                pltpu.make_async_copy(opk_h.at[pl.ds(0, _BCH)], pkn,
                                      sem_l).wait()
                pltpu.make_async_copy(oea0_h.at[pl.ds(0, _BCH)], e0n,
                                      sem_l).wait()
                pltpu.make_async_copy(oea1_h.at[pl.ds(0, _BCH)], e1n,
                                      sem_l).wait()

            def build_sbuf(par):
                sb = sbufs[par]

                def ubody(j, c2):
                    o = j * 16
                    sb[pl.ds(o, 16)] = jnp.minimum(pkn[pl.ds(o, 16)] & 0x3FFF,
                                                   N - 1)
                    return c2
                lax.fori_loop(0, _BCH // 16, ubody, 0)

            def issue_gather(par):
                pltpu.async_copy(b_h.at[sbufs[par]], rowss[par], sem_gs[par])

            def wait_gather(par):
                pltpu.make_async_copy(b_h.at[sbufs[par]], rowss[par],
                                      sem_gs[par]).wait()

            def copy_nxt_to_cur():
                def mv(j, c2):
                    o = j * 16
                    pkc[pl.ds(o, 16)] = pkn[pl.ds(o, 16)]
                    e0c[pl.ds(o, 16)] = e0n[pl.ds(o, 16)]
                    e1c[pl.ds(o, 16)] = e1n[pl.ds(o, 16)]
                    return c2
                lax.fori_loop(0, _BCH // 16, mv, 0)

            w0v = [wbuf[pl.ds(f * 16, 16)] for f in range(8)]
            w1v = [wbuf[pl.ds(128 + f * 16, 16)] for f in range(8)]

            def process(cb, par):
                rows = rowss[par]
                lim = jnp.maximum(jnp.minimum(cnt - cb, _BCH), 0)

                def one_edge(e):
                    pk = pkc[pl.ds(e, 16)][0]
                    a0 = e0c[pl.ds(e, 16)][0]
                    a1 = e1c[pl.ds(e, 16)][0]
                    r = lax.shift_right_logical(pk, 14)
                    ab = r * 128
                    if t == 0:
                        do = r * 16
                        degl[pl.ds(do, 16)] = degl[pl.ds(do, 16)] + 1.0
                    ms = []
                    for f in range(8):
                        row = rows[e, pl.ds(f * 16, 16)]
                        ms.append(row + a0 * w0v[f] + a1 * w1v[f])
                    for f in range(8):
                        o = ab + f * 16
                        m = ms[f]
                        sum_r[pl.ds(o, 16)] = sum_r[pl.ds(o, 16)] + m
                        sq_r[pl.ds(o, 16)] = sq_r[pl.ds(o, 16)] + m * m
                        mn_r[pl.ds(o, 16)] = jnp.minimum(mn_r[pl.ds(o, 16)], m)
                        mx_r[pl.ds(o, 16)] = jnp.maximum(mx_r[pl.ds(o, 16)], m)

                def ebody2(j, _2):
                    e = j * 2
                    one_edge(e)
                    one_edge(e + 1)
                    return 0
                lax.fori_loop(0, lim // 2, ebody2, 0)

                def ebody1(j, _2):
                    one_edge((lim // 2) * 2)
                    return 0
                lax.fori_loop(0, lim & 1, ebody1, 0)

            issue_lists(0)
            wait_lists()
            copy_nxt_to_cur()
            build_sbuf(0)
            issue_gather(0)
            issue_lists(_BCH)

            def gbody(g, _):
                c2 = 2 * g
                for par in (0, 1):
                    c = c2 + par
                    cb = c * _BCH
                    wait_lists()
                    build_sbuf(1 - par)
                    issue_gather(1 - par)
                    wait_gather(par)
                    process(cb, par)
                    copy_nxt_to_cur()
                    issue_lists(cb + 2 * _BCH)
                return 0

            lax.fori_loop(0, ngr, gbody, 0)
            wait_lists()
            wait_gather(0)

            for s, acc in enumerate((sum_r, sq_r, mn_r, mx_r)):
                ob = ((t * 4 + s) * _NV + v) * _RL
                pltpu.sync_copy(acc, out_h.at[pl.ds(ob, _RL)])
            if t == 0:
                pltpu.sync_copy(degl,
                                deg_h.at[pl.ds(v * _ROWS * 16, _ROWS * 16)])


def _mm_kernel(a_ref, w_ref, o_ref):
    o_ref[...] = jnp.dot(a_ref[...], w_ref[...],
                         preferred_element_type=jnp.float32)


def _mm(a, w, bm=2000):
    m, kk = a.shape
    n = w.shape[1]
    return pl.pallas_call(
        _mm_kernel,
        grid=(m // bm,),
        in_specs=[pl.BlockSpec((bm, kk), lambda i: (i, 0)),
                  pl.BlockSpec((kk, n), lambda i: (0, 0))],
        out_specs=pl.BlockSpec((bm, n), lambda i: (i, 0)),
        out_shape=jax.ShapeDtypeStruct((m, n), jnp.float32),
    )(a, w)


def _unshuffle(arr):
    lead = arr.reshape(_NW, 2, _ROWS, -1)
    return lead.transpose(2, 1, 0, 3).reshape(_ROWS * _NV, -1)[:N]


def kernel(x, edge_index, batch, hls_attr, edge_attr, We0, be0, Wpre0_0,
           bpre0_0, Wpost0_0, bpost0_0, Wpre0_1, bpre0_1, Wpost0_1, bpost0_1,
           Wlin0, blin0, We1, be1, Wpre1_0, bpre1_0, Wpost1_0, bpost1_0,
           Wpre1_1, bpre1_1, Wpost1_1, bpost1_1, Wlin1, blin1, Wm0, bm0, Wm1,
           bm1, Wm2, bm2):
    p = dict(
        Wpre0=(Wpre0_0, Wpre0_1), bpre0=(bpre0_0, bpre0_1),
        Wpost0=(Wpost0_0, Wpost0_1), bpost0=(bpost0_0, bpost0_1),
        Wpre1=(Wpre1_0, Wpre1_1), bpre1=(bpre1_0, bpre1_1),
        Wpost1=(Wpost1_0, Wpost1_1), bpost1=(bpost1_0, bpost1_1),
        We0=We0, We1=We1, be0=be0, be1=be1,
        Wlin0=Wlin0, Wlin1=Wlin1, blin0=blin0, blin1=blin1,
    )
    src = edge_index[0].astype(jnp.int32)
    dst = edge_index[1].astype(jnp.int32)
    ea0 = edge_attr[:, 0] + 0.0
    ea1 = edge_attr[:, 1] + 0.0

    opk, oea0, oea1, cnts = _scan_kernel(dst, src, ea0, ea1)
    deg = None

    for l in range(LAYERS):
        Wpre = p[f'Wpre{l}']
        A = jnp.concatenate([Wpre[t][:F] for t in range(T)], axis=1)
        B = jnp.concatenate([Wpre[t][F:2 * F] for t in range(T)], axis=1)
        C = jnp.concatenate([Wpre[t][2 * F:] for t in range(T)], axis=1)
        WeC = p[f'We{l}'] @ C
        beC = p[f'be{l}'] @ C
        bias = jnp.concatenate([p[f'bpre{l}'][t] for t in range(T)]) + beC

        ab = _mm(x, jnp.concatenate([A, B], axis=1))
        a = ab[:, :T * F] + bias
        b0 = ab[:, 2 * F:3 * F] + 0.0
        b1 = ab[:, 3 * F:] + 0.0
        wv = jnp.concatenate(
            [WeC[0, :F], WeC[1, :F], WeC[0, F:], WeC[1, F:]])

        acc, deg_t = _accum_kernel(b0, b1, opk, oea0, oea1, cnts, wv)
        if deg is None:
            deg = _unshuffle(
                deg_t.reshape(_NV, _ROWS, 16)[:, :, 0]).reshape(-1)
            degc = jnp.clip(deg, 1.0, None)
            nonempty = deg > 0.0
            nonemptyf = nonempty.astype(jnp.float32)[:, None]
            inv_degc = (1.0 / degc)[:, None]
            s1 = (jnp.log(degc + 1.0) / _LOG_AVG)[:, None]
            s2 = (_LOG_AVG / jnp.log(degc + 1.0))[:, None]
        st = acc.reshape(2, 4, _NW, 2, _ROWS, 128)
        st = st.transpose(1, 4, 3, 2, 0, 5).reshape(4, _ROWS * _NV, 256)
        st = st[:, :N]
        sum_m, sq_m, mn_m, mx_m = st[0], st[1], st[2], st[3]

        mean_m = sum_m * inv_degc
        msq_m = sq_m * inv_degc
        mean = (a + mean_m) * nonemptyf
        mn = jnp.where(nonempty[:, None], a + mn_m, 0.0)
        mx = jnp.where(nonempty[:, None], a + mx_m, 0.0)
        std = jnp.sqrt(jnp.maximum(msq_m - mean_m * mean_m, 0.0) + 1e-5)

        outs = []
        for t in range(T):
            sl = slice(t * F, (t + 1) * F)
            base = jnp.concatenate(
                [mean[:, sl], mn[:, sl], mx[:, sl], std[:, sl]], axis=1)
            Wpost = p[f'Wpost{l}'][t]
            P0 = Wpost[:F]
            Puvw = jnp.concatenate(
                [Wpost[F:F + 4 * F], Wpost[F + 4 * F:F + 8 * F],
                 Wpost[F + 8 * F:]], axis=1)
            uvw = _mm(base, Puvw)
            out_t = (x @ P0 + uvw[:, :FT] + s1 * uvw[:, FT:2 * FT]
                     + s2 * uvw[:, 2 * FT:] + p[f'bpost{l}'][t])
            outs.append(out_t)
        out = jnp.concatenate(outs, axis=1)
        x = jax.nn.relu(_mm(out, p[f'Wlin{l}']) + p[f'blin{l}'])

    pooled = jax.ops.segment_sum(x, batch, num_segments=G)
    h = jnp.concatenate([pooled, hls_attr], axis=1)
    h = jax.nn.relu(h @ Wm0 + bm0)
    h = jax.nn.relu(h @ Wm1 + bm1)
    return h @ Wm2 + bm2


# speedup vs baseline: 1.0251x; 1.0251x over previous
import functools
import numpy as np
import jax
import jax.numpy as jnp
from jax import lax
from jax.experimental import pallas as pl
from jax.experimental.pallas import tpu as pltpu
from jax.experimental.pallas import tpu_sc as plsc

N, E, F, T, HID, G, HLS, LAYERS = 10000, 320000, 128, 2, 128, 64, 32, 2
FT = HID // T
_DEG_HIST = np.array([0, 100, 500, 1500, 2500, 2400, 1500, 800, 400, 200, 100],
                     dtype=np.float64)
_LOG_AVG = float((np.log(np.arange(len(_DEG_HIST)) + 1.0) * _DEG_HIST).sum()
                 / _DEG_HIST.sum())

_NC, _NS = 2, 16
_NW = _NC * _NS
_NV = 64
_ROWS = 157
_RL = _ROWS * 128
_DPAD = 160
_LCAP = 8192
_FLUSH = 2048
_SCCH = 4000
_BCH = 128
_OBUF = _FLUSH + 512 + 16
_BIG = 3.0e38

_mesh = plsc.VectorSubcoreMesh(core_axis_name="c", subcore_axis_name="s",
                               num_cores=_NC, num_subcores=_NS)


def _wid():
    return lax.axis_index("s") * _NC + lax.axis_index("c")


@functools.partial(
    pl.kernel,
    out_type=(
        jax.ShapeDtypeStruct((_NV * _LCAP,), jnp.int32),
        jax.ShapeDtypeStruct((_NV * _LCAP,), jnp.float32),
        jax.ShapeDtypeStruct((_NV * _LCAP,), jnp.float32),
        jax.ShapeDtypeStruct((_NV * 128,), jnp.int32),
    ),
    mesh=_mesh,
    scratch_types=[
        pltpu.VMEM((_SCCH,), jnp.int32),
        pltpu.VMEM((_SCCH,), jnp.int32),
        pltpu.VMEM((_SCCH,), jnp.float32),
        pltpu.VMEM((_SCCH,), jnp.float32),
        pltpu.VMEM((_OBUF,), jnp.int32),
        pltpu.VMEM((_OBUF,), jnp.float32),
        pltpu.VMEM((_OBUF,), jnp.float32),
        pltpu.VMEM((_OBUF,), jnp.int32),
        pltpu.VMEM((_OBUF,), jnp.float32),
        pltpu.VMEM((_OBUF,), jnp.float32),
        pltpu.VMEM((128,), jnp.int32),
    ],
    compiler_params=pltpu.CompilerParams(needs_layout_passes=False),
)
def _scan_kernel(dst_h, src_h, ea0_h, ea1_h,
                 opk_h, oea0_h, oea1_h, cnt_h,
                 dbuf, sbuf, e0buf, e1buf,
                 pk0, ea0b0, ea1b0, pk1, ea0b1, ea1b1, cstage):
    k = _wid()

    bufs = ((pk0, ea0b0, ea1b0), (pk1, ea0b1, ea1b1))

    def chunk(ci, carry):
        p0, nf0, p1, nf1 = carry
        base = ci * _SCCH
        pltpu.sync_copy(dst_h.at[pl.ds(base, _SCCH)], dbuf)
        pltpu.sync_copy(src_h.at[pl.ds(base, _SCCH)], sbuf)
        pltpu.sync_copy(ea0_h.at[pl.ds(base, _SCCH)], e0buf)
        pltpu.sync_copy(ea1_h.at[pl.ds(base, _SCCH)], e1buf)

        def vbody(i, ps):
            q0, q1 = ps
            off = i * 16
            d = dbuf[pl.ds(off, 16)]
            sub = lax.shift_right_logical(d, 5) & 1
            r = lax.shift_right_logical(d, 6)
            pk = sbuf[pl.ds(off, 16)] | (r << 14)
            e0 = e0buf[pl.ds(off, 16)]
            e1 = e1buf[pl.ds(off, 16)]
            manyk = (d & 31) == k
            m0 = manyk & (sub == 0)
            m1 = manyk & (sub == 1)
            c0 = m0.astype(jnp.int32)
            c1 = m1.astype(jnp.int32)
            i0 = plsc.cumsum(c0)
            i1 = plsc.cumsum(c1)
            x0 = q0 + i0 - c0
            x1 = q1 + i1 - c1
            plsc.store_scatter(pk0, [x0], pk, mask=m0)
            plsc.store_scatter(ea0b0, [x0], e0, mask=m0)
            plsc.store_scatter(ea1b0, [x0], e1, mask=m0)
            plsc.store_scatter(pk1, [x1], pk, mask=m1)
            plsc.store_scatter(ea0b1, [x1], e0, mask=m1)
            plsc.store_scatter(ea1b1, [x1], e1, mask=m1)
            return (q0 + i0[15], q1 + i1[15])

        p0, p1 = lax.fori_loop(0, _SCCH // 16, vbody, (p0, p1))

        def mkflush(sl):
            pkb, e0b, e1b = bufs[sl]

            def do_flush(c):
                p, nf = c
                fb = (2 * k + sl) * _LCAP + nf * _FLUSH
                pltpu.sync_copy(pkb.at[pl.ds(0, _FLUSH)],
                                opk_h.at[pl.ds(fb, _FLUSH)])
                pltpu.sync_copy(e0b.at[pl.ds(0, _FLUSH)],
                                oea0_h.at[pl.ds(fb, _FLUSH)])
                pltpu.sync_copy(e1b.at[pl.ds(0, _FLUSH)],
                                oea1_h.at[pl.ds(fb, _FLUSH)])
                rem = p - _FLUSH

                def mv(j, c2):
                    o = j * 16
                    pkb[pl.ds(o, 16)] = pkb[pl.ds(_FLUSH + o, 16)]
                    e0b[pl.ds(o, 16)] = e0b[pl.ds(_FLUSH + o, 16)]
                    e1b[pl.ds(o, 16)] = e1b[pl.ds(_FLUSH + o, 16)]
                    return c2
                lax.fori_loop(0, (rem + 15) // 16, mv, 0)
                return rem, nf + 1
            return do_flush

        p0, nf0 = lax.cond(p0 >= _FLUSH, mkflush(0), lambda c: c, (p0, nf0))
        p1, nf1 = lax.cond(p1 >= _FLUSH, mkflush(1), lambda c: c, (p1, nf1))
        return (p0, nf0, p1, nf1)

    p0, nf0, p1, nf1 = lax.fori_loop(0, E // _SCCH, chunk, (0, 0, 0, 0))

    for sl, (p, nf) in enumerate(((p0, nf0), (p1, nf1))):
        pkb, e0b, e1b = bufs[sl]
        fb = (2 * k + sl) * _LCAP + nf * _FLUSH
        pltpu.sync_copy(pkb.at[pl.ds(0, _FLUSH)], opk_h.at[pl.ds(fb, _FLUSH)])
        pltpu.sync_copy(e0b.at[pl.ds(0, _FLUSH)], oea0_h.at[pl.ds(fb, _FLUSH)])
        pltpu.sync_copy(e1b.at[pl.ds(0, _FLUSH)], oea1_h.at[pl.ds(fb, _FLUSH)])
        total = nf * _FLUSH + p
        for j in range(8):
            cstage[pl.ds(j * 16, 16)] = jnp.full((16,), total, jnp.int32)
        pltpu.sync_copy(cstage, cnt_h.at[pl.ds((2 * k + sl) * 128, 128)])


@functools.partial(
    pl.kernel,
    out_type=(jax.ShapeDtypeStruct((2 * 4 * _NV * _RL,), jnp.float32),
              jax.ShapeDtypeStruct((_NV * _ROWS * 16,), jnp.float32)),
    mesh=_mesh,
    scratch_types=[
        pltpu.VMEM((_RL,), jnp.float32),
        pltpu.VMEM((_RL,), jnp.float32),
        pltpu.VMEM((_RL,), jnp.float32),
        pltpu.VMEM((_RL,), jnp.float32),
        pltpu.VMEM((_BCH,), jnp.int32),
        pltpu.VMEM((_BCH,), jnp.int32),
        pltpu.VMEM((_BCH + 16,), jnp.int32),
        pltpu.VMEM((_BCH + 16,), jnp.float32),
        pltpu.VMEM((_BCH + 16,), jnp.float32),
        pltpu.VMEM((_BCH,), jnp.int32),
        pltpu.VMEM((_BCH,), jnp.float32),
        pltpu.VMEM((_BCH,), jnp.float32),
        pltpu.VMEM((_BCH, 128), jnp.float32),
        pltpu.VMEM((_BCH, 128), jnp.float32),
        pltpu.VMEM((256,), jnp.float32),
        pltpu.VMEM((128,), jnp.int32),
        pltpu.VMEM((_ROWS * 16,), jnp.float32),
        pltpu.SemaphoreType.DMA,
        pltpu.SemaphoreType.DMA,
        pltpu.SemaphoreType.DMA,
    ],
    compiler_params=pltpu.CompilerParams(needs_layout_passes=False),
)
def _accum_kernel(b0_h, b1_h, opk_h, oea0_h, oea1_h, cnt_h, wv_h, out_h,
                  deg_h, sum_r, sq_r, mn_r, mx_r, sbuf0, sbuf1, pkc, e0c, e1c,
                  pkn, e0n, e1n, rows0, rows1, wbuf, cstage, degl,
                  sem_l, sem_g0, sem_g1, *_unused):
    k = _wid()
    zf = jnp.zeros((16,), jnp.float32)
    bigf = jnp.full((16,), _BIG, jnp.float32)
    sbufs = (sbuf0, sbuf1)
    rowss = (rows0, rows1)
    sem_gs = (sem_g0, sem_g1)
    maxlb = _LCAP - _BCH

    for t, b_h in enumerate((b0_h, b1_h)):
        pltpu.sync_copy(wv_h.at[pl.ds(t * 256, 256)], wbuf)
        for sl in range(2):
            v = 2 * k + sl
            vbase = v * _LCAP
            pltpu.sync_copy(cnt_h.at[pl.ds(v * 128, 128)], cstage)
            cnt = cstage[pl.ds(0, 16)][0]
            nch = (cnt + _BCH - 1) // _BCH
            ngr = (nch + 1) // 2

            def initb(j, c):
                o = j * 16
                sum_r[pl.ds(o, 16)] = zf
                sq_r[pl.ds(o, 16)] = zf
                mn_r[pl.ds(o, 16)] = bigf
                mx_r[pl.ds(o, 16)] = -bigf
                return c
            lax.fori_loop(0, _RL // 16, initb, 0)
            if t == 0:
                def initd(j, c):
                    degl[pl.ds(j * 16, 16)] = zf
                    return c
                lax.fori_loop(0, _ROWS, initd, 0)

            def issue_lists(cb):
                lb = vbase + jnp.minimum(cb, maxlb)
                pltpu.async_copy(opk_h.at[pl.ds(lb, _BCH)], pkn, sem_l)
                pltpu.async_copy(oea0_h.at[pl.ds(lb, _BCH)], e0n, sem_l)
                pltpu.async_copy(oea1_h.at[pl.ds(lb, _BCH)], e1n, sem_l)

            def wait_lists():
                pltpu.make_async_copy(opk_h.at[pl.ds(0, _BCH)], pkn,
                                      sem_l).wait()
                pltpu.make_async_copy(oea0_h.at[pl.ds(0, _BCH)], e0n,
                                      sem_l).wait()
                pltpu.make_async_copy(oea1_h.at[pl.ds(0, _BCH)], e1n,
                                      sem_l).wait()

            def build_sbuf(par):
                sb = sbufs[par]

                def ubody(j, c2):
                    o = j * 16
                    sb[pl.ds(o, 16)] = jnp.minimum(pkn[pl.ds(o, 16)] & 0x3FFF,
                                                   N - 1)
                    return c2
                lax.fori_loop(0, _BCH // 16, ubody, 0)

            def issue_gather(par):
                pltpu.async_copy(b_h.at[sbufs[par]], rowss[par], sem_gs[par])

            def wait_gather(par):
                pltpu.make_async_copy(b_h.at[sbufs[par]], rowss[par],
                                      sem_gs[par]).wait()

            def copy_nxt_to_cur():
                def mv(j, c2):
                    o = j * 16
                    pkc[pl.ds(o, 16)] = pkn[pl.ds(o, 16)]
                    e0c[pl.ds(o, 16)] = e0n[pl.ds(o, 16)]
                    e1c[pl.ds(o, 16)] = e1n[pl.ds(o, 16)]
                    return c2
                lax.fori_loop(0, _BCH // 16, mv, 0)

            w0v = [wbuf[pl.ds(f * 16, 16)] for f in range(8)]
            w1v = [wbuf[pl.ds(128 + f * 16, 16)] for f in range(8)]

            def process(cb, par):
                rows = rowss[par]
                lim = jnp.maximum(jnp.minimum(cnt - cb, _BCH), 0)

                def one_edge(e):
                    pk = pkc[pl.ds(e, 16)][0]
                    a0 = e0c[pl.ds(e, 16)][0]
                    a1 = e1c[pl.ds(e, 16)][0]
                    r = lax.shift_right_logical(pk, 14)
                    ab = r * 128
                    if t == 0:
                        do = r * 16
                        degl[pl.ds(do, 16)] = degl[pl.ds(do, 16)] + 1.0
                    ms = []
                    for f in range(8):
                        row = rows[e, pl.ds(f * 16, 16)]
                        ms.append(row + a0 * w0v[f] + a1 * w1v[f])
                    for f in range(8):
                        o = ab + f * 16
                        m = ms[f]
                        sum_r[pl.ds(o, 16)] = sum_r[pl.ds(o, 16)] + m
                        sq_r[pl.ds(o, 16)] = sq_r[pl.ds(o, 16)] + m * m
                        mn_r[pl.ds(o, 16)] = jnp.minimum(mn_r[pl.ds(o, 16)], m)
                        mx_r[pl.ds(o, 16)] = jnp.maximum(mx_r[pl.ds(o, 16)], m)

                def ebody2(j, _2):
                    e = j * 2
                    one_edge(e)
                    one_edge(e + 1)
                    return 0
                lax.fori_loop(0, lim // 2, ebody2, 0)

                def ebody1(j, _2):
                    one_edge((lim // 2) * 2)
                    return 0
                lax.fori_loop(0, lim & 1, ebody1, 0)

            issue_lists(0)
            wait_lists()
            copy_nxt_to_cur()
            build_sbuf(0)
            issue_gather(0)
            issue_lists(_BCH)

            def gbody(g, _):
                c2 = 2 * g
                for par in (0, 1):
                    c = c2 + par
                    cb = c * _BCH
                    wait_lists()
                    build_sbuf(1 - par)
                    issue_gather(1 - par)
                    wait_gather(par)
                    process(cb, par)
                    copy_nxt_to_cur()
                    issue_lists(cb + 2 * _BCH)
                return 0

            lax.fori_loop(0, ngr, gbody, 0)
            wait_lists()
            wait_gather(0)

            for s, acc in enumerate((sum_r, sq_r, mn_r, mx_r)):
                ob = ((t * 4 + s) * _NV + v) * _RL
                pltpu.sync_copy(acc, out_h.at[pl.ds(ob, _RL)])
            if t == 0:
                pltpu.sync_copy(degl,
                                deg_h.at[pl.ds(v * _ROWS * 16, _ROWS * 16)])


def _mm_kernel(a_ref, w_ref, o_ref):
    o_ref[...] = jnp.dot(a_ref[...], w_ref[...],
                         preferred_element_type=jnp.float32)


def _mm(a, w, bm=2000):
    m, kk = a.shape
    n = w.shape[1]
    return pl.pallas_call(
        _mm_kernel,
        grid=(m // bm,),
        in_specs=[pl.BlockSpec((bm, kk), lambda i: (i, 0)),
                  pl.BlockSpec((kk, n), lambda i: (0, 0))],
        out_specs=pl.BlockSpec((bm, n), lambda i: (i, 0)),
        out_shape=jax.ShapeDtypeStruct((m, n), jnp.float32),
    )(a, w)


def _unshuffle(arr):
    lead = arr.reshape(_NW, 2, _ROWS, -1)
    return lead.transpose(2, 1, 0, 3).reshape(_ROWS * _NV, -1)[:N]


_PBM = 2000


def _pool_mlp_kernel(b_ref, x_ref, hls_ref, w0_ref, b0_ref, w1_ref, b1_ref,
                     w2_ref, b2_ref, o_ref, acc_ref):
    i = pl.program_id(0)

    @pl.when(i == 0)
    def _():
        acc_ref[...] = jnp.zeros_like(acc_ref)

    seg = jax.lax.broadcasted_iota(jnp.int32, (G, _PBM), 0)
    oh = (b_ref[0, 0, :][None, :] == seg).astype(jnp.float32)
    acc_ref[...] += jnp.dot(oh, x_ref[...], preferred_element_type=jnp.float32)

    @pl.when(i == N // _PBM - 1)
    def _():
        h = jnp.concatenate([acc_ref[...], hls_ref[...][:, :HLS]], axis=1)
        h = jax.nn.relu(jnp.dot(h, w0_ref[...],
                                preferred_element_type=jnp.float32)
                        + b0_ref[0, :][None, :])
        h = jax.nn.relu(jnp.dot(h, w1_ref[...],
                                preferred_element_type=jnp.float32)
                        + b1_ref[0, :][None, :])
        o_ref[...] = (jnp.dot(h, w2_ref[...],
                              preferred_element_type=jnp.float32)
                      + b2_ref[0, :][None, :])


def _pool_mlp(batch3, x, hlsp, w0p, b0p, w1p, b1p, w2p, b2p):
    cst = lambda i: (0, 0)
    return pl.pallas_call(
        _pool_mlp_kernel,
        grid=(N // _PBM,),
        in_specs=[pl.BlockSpec((1, 1, _PBM), lambda i: (i, 0, 0)),
                  pl.BlockSpec((_PBM, HID), lambda i: (i, 0)),
                  pl.BlockSpec((G, 128), cst),
                  pl.BlockSpec((HID + HLS, 128), cst),
                  pl.BlockSpec((8, 128), cst),
                  pl.BlockSpec((128, 128), cst),
                  pl.BlockSpec((8, 128), cst),
                  pl.BlockSpec((128, 128), cst),
                  pl.BlockSpec((8, 128), cst)],
        out_specs=pl.BlockSpec((G, 128), cst),
        out_shape=jax.ShapeDtypeStruct((G, 128), jnp.float32),
        scratch_shapes=[pltpu.VMEM((G, HID), jnp.float32)],
    )(batch3, x, hlsp, w0p, b0p, w1p, b1p, w2p, b2p)


def kernel(x, edge_index, batch, hls_attr, edge_attr, We0, be0, Wpre0_0,
           bpre0_0, Wpost0_0, bpost0_0, Wpre0_1, bpre0_1, Wpost0_1, bpost0_1,
           Wlin0, blin0, We1, be1, Wpre1_0, bpre1_0, Wpost1_0, bpost1_0,
           Wpre1_1, bpre1_1, Wpost1_1, bpost1_1, Wlin1, blin1, Wm0, bm0, Wm1,
           bm1, Wm2, bm2):
    p = dict(
        Wpre0=(Wpre0_0, Wpre0_1), bpre0=(bpre0_0, bpre0_1),
        Wpost0=(Wpost0_0, Wpost0_1), bpost0=(bpost0_0, bpost0_1),
        Wpre1=(Wpre1_0, Wpre1_1), bpre1=(bpre1_0, bpre1_1),
        Wpost1=(Wpost1_0, Wpost1_1), bpost1=(bpost1_0, bpost1_1),
        We0=We0, We1=We1, be0=be0, be1=be1,
        Wlin0=Wlin0, Wlin1=Wlin1, blin0=blin0, blin1=blin1,
    )
    src = edge_index[0].astype(jnp.int32)
    dst = edge_index[1].astype(jnp.int32)
    ea0 = edge_attr[:, 0] + 0.0
    ea1 = edge_attr[:, 1] + 0.0

    opk, oea0, oea1, cnts = _scan_kernel(dst, src, ea0, ea1)
    deg = None

    for l in range(LAYERS):
        Wpre = p[f'Wpre{l}']
        A = jnp.concatenate([Wpre[t][:F] for t in range(T)], axis=1)
        B = jnp.concatenate([Wpre[t][F:2 * F] for t in range(T)], axis=1)
        C = jnp.concatenate([Wpre[t][2 * F:] for t in range(T)], axis=1)
        WeC = p[f'We{l}'] @ C
        beC = p[f'be{l}'] @ C
        bias = jnp.concatenate([p[f'bpre{l}'][t] for t in range(T)]) + beC

        ab = _mm(x, jnp.concatenate([A, B], axis=1))
        a = ab[:, :T * F] + bias
        b0 = ab[:, 2 * F:3 * F] + 0.0
        b1 = ab[:, 3 * F:] + 0.0
        wv = jnp.concatenate(
            [WeC[0, :F], WeC[1, :F], WeC[0, F:], WeC[1, F:]])

        acc, deg_t = _accum_kernel(b0, b1, opk, oea0, oea1, cnts, wv)
        if deg is None:
            deg = _unshuffle(
                deg_t.reshape(_NV, _ROWS, 16)[:, :, 0]).reshape(-1)
            degc = jnp.clip(deg, 1.0, None)
            nonempty = deg > 0.0
            nonemptyf = nonempty.astype(jnp.float32)[:, None]
            inv_degc = (1.0 / degc)[:, None]
            s1 = (jnp.log(degc + 1.0) / _LOG_AVG)[:, None]
            s2 = (_LOG_AVG / jnp.log(degc + 1.0))[:, None]
        st = acc.reshape(2, 4, _NW, 2, _ROWS, 128)
        st = st.transpose(1, 4, 3, 2, 0, 5).reshape(4, _ROWS * _NV, 256)
        st = st[:, :N]
        sum_m, sq_m, mn_m, mx_m = st[0], st[1], st[2], st[3]

        mean_m = sum_m * inv_degc
        msq_m = sq_m * inv_degc
        mean = (a + mean_m) * nonemptyf
        mn = jnp.where(nonempty[:, None], a + mn_m, 0.0)
        mx = jnp.where(nonempty[:, None], a + mx_m, 0.0)
        std = jnp.sqrt(jnp.maximum(msq_m - mean_m * mean_m, 0.0) + 1e-5)

        P0b = jnp.concatenate([p[f'Wpost{l}'][t][:F] for t in range(T)],
                              axis=1)
        xp = _mm(x, P0b)
        outs = []
        for t in range(T):
            sl = slice(t * F, (t + 1) * F)
            base = jnp.concatenate(
                [mean[:, sl], mn[:, sl], mx[:, sl], std[:, sl]], axis=1)
            Wpost = p[f'Wpost{l}'][t]
            Puvw = jnp.concatenate(
                [Wpost[F:F + 4 * F], Wpost[F + 4 * F:F + 8 * F],
                 Wpost[F + 8 * F:]], axis=1)
            uvw = _mm(base, Puvw)
            out_t = (xp[:, t * FT:(t + 1) * FT] + uvw[:, :FT]
                     + s1 * uvw[:, FT:2 * FT]
                     + s2 * uvw[:, 2 * FT:] + p[f'bpost{l}'][t])
            outs.append(out_t)
        out = jnp.concatenate(outs, axis=1)
        x = jax.nn.relu(_mm(out, p[f'Wlin{l}']) + p[f'blin{l}'])

    batch3 = batch.astype(jnp.int32).reshape(N // _PBM, 1, _PBM)
    hlsp = jnp.pad(hls_attr, ((0, 0), (0, 128 - HLS)))
    w0p = jnp.pad(Wm0, ((0, 0), (0, 64)))
    b0p = jnp.zeros((8, 128), jnp.float32).at[0, :64].set(bm0)
    w1p = jnp.pad(Wm1, ((0, 64), (0, 64)))
    b1p = jnp.zeros((8, 128), jnp.float32).at[0, :64].set(bm1)
    w2p = jnp.pad(Wm2, ((0, 64), (0, 127)))
    b2p = jnp.zeros((8, 128), jnp.float32).at[0, :1].set(bm2)
    out = _pool_mlp(batch3, x, hlsp, w0p, b0p, w1p, b1p, w2p, b2p)
    return out[:, :1]
